# baseline (speedup 1.0000x reference)
import functools

import jax
import jax.numpy as jnp
import numpy as np
from jax.experimental import pallas as pl

_M = 10
_NF = 256
_P = 256
_K = 0.01
_T = _M // 2
_HI = jax.lax.Precision.HIGHEST
_LO = jax.lax.Precision.DEFAULT

_MU0 = np.asarray(
    jax.random.normal(jax.random.key(42), (_P, _M + 1), dtype=jnp.float32),
    dtype=np.float32).T.copy()


def _dg(row, mat, precision=_HI):
    return jax.lax.dot_general(
        row, mat, (((1,), (1,)), ((), ())),
        precision=precision, preferred_element_type=jnp.float32)


def _s2v_kernel(z_ref, xp_ref, b1_ref, th1_ref, th2_ref, th3_ref, th2c_ref,
                th3c_ref, th4a_ref, th4b_ref, mu0_ref, out_ref):
    z = z_ref[0, 0]
    b1 = b1_ref[0, 0]
    b2 = b1 * b1

    xp = xp_ref[...]
    gram = _dg(xp, xp)

    rid = jax.lax.broadcasted_iota(jnp.int32, (16, 16), 0)
    cid = jax.lax.broadcasted_iota(jnp.int32, (16, 16), 1)
    nb_mask = (cid < _M) & (cid != rid)
    w = gram + b2
    pos = jnp.sum(jnp.where(nb_mask, jnp.maximum(w, 0.0), 0.0), axis=1,
                  keepdims=True)
    neg = jnp.sum(jnp.where(nb_mask, jnp.maximum(-w, 0.0), 0.0), axis=1,
                  keepdims=True)

    g9 = jnp.sum(jnp.where((rid == _M) & (cid == 9), gram, 0.0))
    g8 = jnp.sum(jnp.where((rid == _M) & (cid == 8), gram, 0.0))
    vcol = jax.lax.broadcasted_iota(jnp.int32, (16, 1), 0)
    wlast = jnp.where(vcol == 9, g8, g9)

    arows = jax.lax.broadcasted_iota(jnp.int32, (16, 1), 0)
    a_z = jnp.sum(jnp.where(arows == z, xp, 0.0), axis=0, keepdims=True)
    t1 = _K * _dg(a_z, th1_ref[...])

    th3 = th3_ref[...]
    th3c = th3c_ref[...]
    base = (t1
            + _K * (pos * jnp.maximum(th3, 0.0)
                    + neg * jnp.maximum(-th3, 0.0))
            + _K * (jnp.maximum(wlast, 0.0) * jnp.maximum(th3c, 0.0)
                    + jnp.maximum(-wlast, 0.0) * jnp.maximum(-th3c, 0.0)))

    th2 = th2_ref[...]
    th2t_lo = jnp.transpose(th2[:, :128])
    th2_hi = th2[:, 128:]
    th2c = th2c_ref[...]

    def t2mv(u):
        t2m = _dg(u[:, 128:], th2_hi, _LO)
        u_col = jnp.reshape(u[:, :128], (128, 1))
        t2v = jnp.sum(th2t_lo * u_col, axis=0, keepdims=True)
        return _K * (t2m + t2v)
    cols = [mu0_ref[v:v + 1, :] for v in range(_M + 1)]
    s_all = functools.reduce(jnp.add, cols[:_M])
    for _ in range(_T):
        bc = base + _K * _dg(cols[_M], th2c)
        sv = s_all - cols[0]
        for v in range(_M):
            x = jnp.maximum(bc[v:v + 1, :] + t2mv(sv), 0.0)
            s_all = sv + x
            cols[v] = x
            if v + 1 < _M:
                sv = s_all - cols[v + 1]
        cols[_M] = jnp.maximum(bc[_M:_M + 1, :] + t2mv(s_all), 0.0)

    r4a = _dg(s_all, th4a_ref[...])
    mu_z = functools.reduce(
        jnp.add,
        [jnp.where(z == v, 1.0, 0.0) * cols[v] for v in range(_M)])
    r4b = _dg(mu_z, th4b_ref[...])
    out_ref[...] = jnp.concatenate([r4a, r4b], axis=1)


@jax.jit
def kernel(A, b, c, z, theta1, theta2, theta3, theta2c, theta3c, theta4a,
           theta4b):
    a0 = A[0]
    xp = jnp.concatenate(
        [a0, c, jnp.zeros((5, _NF), jnp.float32)], axis=0)
    b1 = b[:, 1, 0].reshape(1, 1)
    zz = z.reshape(1, 1)
    mu0 = jnp.asarray(_MU0)
    out = pl.pallas_call(
        _s2v_kernel,
        out_shape=jax.ShapeDtypeStruct((1, 2 * _P), jnp.float32),
    )(zz, xp, b1, theta1, theta2, theta3.reshape(1, _P),
      theta2c, theta3c.reshape(1, _P), theta4a, theta4b, mu0)
    return out


# speedup vs baseline: 1.1077x; 1.1077x over previous
import functools

import jax
import jax.numpy as jnp
import numpy as np
from jax.experimental import pallas as pl

_M = 10
_NF = 256
_P = 256
_K = 0.01
_T = _M // 2
_HI = jax.lax.Precision.HIGHEST
_LO = jax.lax.Precision.DEFAULT

_MU0 = np.asarray(
    jax.random.normal(jax.random.key(42), (_P, _M + 1), dtype=jnp.float32),
    dtype=np.float32).T.copy()


def _dg(row, mat, precision=_HI):
    return jax.lax.dot_general(
        row, mat, (((1,), (1,)), ((), ())),
        precision=precision, preferred_element_type=jnp.float32)


def _s2v_kernel(z_ref, a_ref, b_ref, c_ref, th1_ref, th2_ref, th3_ref,
                th2c_ref, th3c_ref, th4a_ref, th4b_ref, mu0_ref, out_ref):
    z = z_ref[0, 0]
    b1 = b_ref[1, 0]
    b2 = b1 * b1

    xp = jnp.concatenate(
        [a_ref[...], c_ref[...], jnp.zeros((5, _NF), jnp.float32)], axis=0)
    gram = _dg(xp, xp)

    rid = jax.lax.broadcasted_iota(jnp.int32, (16, 16), 0)
    cid = jax.lax.broadcasted_iota(jnp.int32, (16, 16), 1)
    nb_mask = (cid < _M) & (cid != rid)
    w = gram + b2
    pos = jnp.sum(jnp.where(nb_mask, jnp.maximum(w, 0.0), 0.0), axis=1,
                  keepdims=True)
    neg = jnp.sum(jnp.where(nb_mask, jnp.maximum(-w, 0.0), 0.0), axis=1,
                  keepdims=True)

    g9 = jnp.sum(jnp.where((rid == _M) & (cid == 9), gram, 0.0))
    g8 = jnp.sum(jnp.where((rid == _M) & (cid == 8), gram, 0.0))
    vcol = jax.lax.broadcasted_iota(jnp.int32, (16, 1), 0)
    wlast = jnp.where(vcol == 9, g8, g9)

    arows = jax.lax.broadcasted_iota(jnp.int32, (16, 1), 0)
    a_z = jnp.sum(jnp.where(arows == z, xp, 0.0), axis=0, keepdims=True)
    t1 = _K * _dg(a_z, th1_ref[...])

    th3 = th3_ref[...]
    th3c = th3c_ref[...]
    base = (t1
            + _K * (pos * jnp.maximum(th3, 0.0)
                    + neg * jnp.maximum(-th3, 0.0))
            + _K * (jnp.maximum(wlast, 0.0) * jnp.maximum(th3c, 0.0)
                    + jnp.maximum(-wlast, 0.0) * jnp.maximum(-th3c, 0.0)))

    th2 = th2_ref[...]
    th2t_lo = jnp.transpose(th2[:, :128])
    th2_hi = th2[:, 128:]
    th2c = th2c_ref[...]

    def t2mv(u):
        t2m = _dg(u[:, 128:], th2_hi, _LO)
        u_col = jnp.reshape(u[:, :128], (128, 1))
        t2v = jnp.sum(th2t_lo * u_col, axis=0, keepdims=True)
        return _K * (t2m + t2v)
    cols = [mu0_ref[v:v + 1, :] for v in range(_M + 1)]
    s_all = functools.reduce(jnp.add, cols[:_M])
    for _ in range(_T):
        bc = base + _K * _dg(cols[_M], th2c)
        sv = s_all - cols[0]
        for v in range(_M):
            x = jnp.maximum(bc[v:v + 1, :] + t2mv(sv), 0.0)
            s_all = sv + x
            cols[v] = x
            if v + 1 < _M:
                sv = s_all - cols[v + 1]
        cols[_M] = jnp.maximum(bc[_M:_M + 1, :] + t2mv(s_all), 0.0)

    r4a = _dg(s_all, th4a_ref[...])
    mu_z = functools.reduce(
        jnp.add,
        [jnp.where(z == v, 1.0, 0.0) * cols[v] for v in range(_M)])
    r4b = _dg(mu_z, th4b_ref[...])
    out_ref[...] = jnp.concatenate([r4a, r4b], axis=1)


@jax.jit
def kernel(A, b, c, z, theta1, theta2, theta3, theta2c, theta3c, theta4a,
           theta4b):
    out = pl.pallas_call(
        _s2v_kernel,
        out_shape=jax.ShapeDtypeStruct((1, 2 * _P), jnp.float32),
    )(z.reshape(1, 1), A.reshape(_M, _NF), b.reshape(2, 1), c,
      theta1, theta2, theta3.reshape(1, _P),
      theta2c, theta3c.reshape(1, _P), theta4a, theta4b, jnp.asarray(_MU0))
    return out


# speedup vs baseline: 1.1133x; 1.0051x over previous
import functools

import jax
import jax.numpy as jnp
import numpy as np
from jax.experimental import pallas as pl

_M = 10
_NF = 256
_P = 256
_K = 0.01
_T = _M // 2
_HI = jax.lax.Precision.HIGHEST
_LO = jax.lax.Precision.DEFAULT

import base64 as _b64, zlib as _zlib
_MU0_B64 = (
    "eNoN14c7lm0YBnBJ9sqqRCGliGyR575eWQ2VUEghldHQkhZR2WUle0QZlYysrOe+XiOhkqxK0VdE"
    "2luDfP0L13Fc5/k7ZZ+P1te9iocTWj9xxEUTtu5D8sjZkC5X5kJ3Zhb2vJ3DKVcJRpm0/8j7pRb0"
    "5JcALBZfBnBnFnce3zUyYjtGUg2kwGfPLfDwOYsHsnaio3gRHKpUpo/5M0Dk+yh9e10XnQL+MPe1"
    "52Ki3kNUJJVUk9WnqpOWYC/9nbqM7qb9uudxx/woNKxSQdWrm8nhsDSysCYA+dXlQCBmM7pW7cQQ"
    "bxs4ULgSuoO0kT9WktN22BGKHnQRiV8pYHfUDxt+3QKeCWNQ4JdHsal4eDrmi/sFp2PApiB4VVWD"
    "Yj9mc973vSGC/pFw43UWNofmY17tN/K5mp+c/CMJ24/p4Vz/FCbWeDnu+qmLN0wSSKL7FybPuIWA"
    "1z5o9JQmjZ1luLIujgp816S2qWNMdd5ltJ/MItkieWx/Xil2qjaA97l8dLi3HV9/W4kFimFU7mcv"
    "CbJxx7ToEZJgJsYdMuzG7r5hjBraTy11e8kqxQRyQCMYtleI0vSkBur4zAh/mdZgsksZRA4qcH/e"
    "yIEDvBrgmBdCHIJ8ccG1NTiYmgRXDtaTjqeXQW3cE6w/fMM283Dib3iBylydwW1+34DKkqKovfkJ"
    "eWX5hjXWW8R9lhGMIdHK4BS1FCNGknBetjY2SU6B+rN1uNRTETq7yyH7v310x54s+CIyDTtOxxAb"
    "USN0/tFEdYfMoWv/Qkxbc420BDvAH/e4Fc9cOuiPP+vo+sSd2LilnvxskeQoS5RRr+5rNElpFZG/"
    "rMeJyA/Gttu/qGVlPjv+04q89J7G3T57HD0Uf7IyFZa04LMkfnakIFdiAc12XTQ8xQk6LxvizGOS"
    "eGptEulPmwXpKw1w0TF+EAooQqmSWCzI1ACl10nYLbIfh148Rcvz10wHlJ2RfTMEFRpV9PZYHNTu"
    "mwVX5l2hG7OPYHUNwc1zvwF7whhvnQglk5dvgU6xC6ZsHMAATTkS82sWTjh/gcdR9TjHVQgfJuWQ"
    "G+37sdNXjPNp+yGmf+MF6Diig3azOfBYP5TWSq7HrxtXwu3NbVjUuxA9K81ReKIGS+6ngs/VXjx1"
    "QR7HZivC+OcOONf+gH7gHUdf8TCMqG8HvQvO1NWiidzREUJ102Pg+XkVyetVxYhpalDf1kjCfU+S"
    "4luq+HmqCd7vBLIyKxqSxu/W2uZ9gsBtmlzbhY+J18xitqWeB4IyMnDODQ98ElNDUho3MtGtneDa"
    "YAvpObHQ7lpIc1MGICfAAxZfuUb9332j659tYLzlyiCO9x5kJ8lwVb7ywmuNjyRMogY2eleRrTOG"
    "GNYuHBvOHaHxenXoqAZUcmIFpHhnMm+q9eDOF1nqvdGQHtcoohFqSpz0My5k8edahu85S9fFF8HC"
    "tB1we/8w8R91xyqf15BctBrj9M0hMl0WLoT8B38W/cuHHg/YY3CVZDcPYw7vA1zCvQcy2Rto9fgU"
    "lRjzJK+7RWHB/iD8essGb16MQPOd72BB+A28YpxGrOhXnDLRgCjdfrp6UyPJFP5NizwNIb1Plfwc"
    "z4eD9tPgnOFplKrfx4zuNSM3hjawO2p2wfekZPjl2Ayt1YvYJeYTrNHzDRgSug4XrVuP3zlv6YKO"
    "NjTvH6M7ltpAE9XD+u5LdFl7Mek/mArrRIQ4yhGKoO1xHyeW5+CCnCrofVMDddWzMdFpFyj4Tefw"
    "hD2CzQLi4NDhgE/VfRmO/y7oPOUMHf2V9IrgNE5p6QyoCUjGReEM6MwX4A6UXkH578m47b/VIJY6"
    "jSujcA6ucnJhU9Z8MEhYie0CT9j9A60YtnAn+GnE4qqoRDy0rZlCvCRYZbxB41peMqc+Gtsa4xi3"
    "pMuEH+So71J7EJowwHf6K9Bozw+cKSsAyo+FSU/odG74lBChdRdxS4s48LtEkr04E3yMV+IVES3y"
    "HxOG2WnqXEu/UUbbjoebr/6KtT+0Eori6ukarQfU5kAuGP3NoUsWm8FdG38QP1VL07ZOo0EtBoS6"
    "niCdhh74OO4R+3INkhyDbPjTLFH/lSMMOwo3oD59T231SqlHkSWWRP6mv7834+gIP+rEp9AVO3cT"
    "84WLIVNPA+XuK2FBZRwRvaQEA7eKSJPkIHpKx8DzbxJg6r8G2CRpzjZ3abx41giXVDCwWzwHXncd"
    "wsd2e5EVmMZZfzgMqobUyJBKMLz/oQSu5vz4mZfChiW3yYw5O8B04xI8/SWFPMuV4QrvvIsfr5Sb"
    "OkQvIqMiaZhqoAR56WUk8Z45XiyrRV0vC/jUE4Ondetp42NFOuhnTyTgBQmfuQDKlg8TSYtPsHqV"
    "Cq7ZF4L7xW9A9Yc2qHOrBBOyBN9UrmJ/tc3AA/UzcI9OG/5w7mP04y7gX74gWGV1jF5bPILFUiVY"
    "eOMM+lp0oeH7zWzv+2jCH6fKEfp6HZv7lwHfzat0hW0DNTZMobXNc2B54Auwc79HF55eyu0MGwD4"
    "rowBhX6sskg54dXSwsgmN2zrW4gCOzaQrCsiOAY2rIthHH4Y0cZgvkYwC3zJ3jazxRzpCjbdsBHT"
    "zrSQWNVSbG+OglRZO9jumUwWuUSB9Gt/uly0jEqUDZO13Q+pz5I8xiNaAYrum0C3yga4dq2VxvaM"
    "kzyx2RxONg9XfXYArLltixzZ8+zLhzPo2ucPMLLdB34ftYWfHl5wYN1pdle1BQ1dfxq+LxHhvl9W"
    "D3sKn0NmSh49ulkT2zt5cGnaYlDwDcPis4CMwHI8aqgCDpf1UOhAJ2LEVxLg64qb+jmQF8WHCwcX"
    "wqGLI6T9dzisoetB/vpZzGuwpAkFf8FKmaBcdy5ZND5JNf0HwXHbS7LPrYMmiHeQopXNEPzvX+f3"
    "CHFs5HLxC1uKNXsOkZcvrsIJp/0wu+k0s3erOFdyrAIzpN+D3FcBOu5WQFKndZJegyiyo/4ccfBR"
    "Z9V/C9Ozws1w2GzrCo9vCviRVpCb349QiZnGSNP8qaZ1KIkOmgWPVeeil/49dJPdhGEyvczRr23o"
    "G2UGOq9nwsf9g3SRySXoO3IGPi7jA4cSPjxauRYfHxHmLlFXxM9yFajwUdKk9Xo1rLxfDFHUHN5V"
    "RhAm3hUabBPw+Jt5+C7xDHVQvArndROZu6HC2HxFitPBqwl7J13BKt2CNmnPIMrKs3Hw5mOym/gx"
    "IyZh0Ln0Ih06/4K+s3DAWwOZkBz+E+Qq/9Bt/IPY97MO73pn1wcyTpDPL4IbYRdsabWDjh/NuOr2"
    "DPJ02lp8tWeYCl71r3/CY0GUpedjmOQZsjWxyrQ6oARi3xWT4FOmcNa9Hn5kdJDNcm6sROMu+F2Z"
    "zDgPvYW//fOQm7CV7mqNJsqaEvg0NAIeHVFC/5uH8UueIO7V4sPO19KY8/QarI39gn2BTiDK50JO"
    "abuD/oY88JCXx5zeXfBGzRN5F4kS1cPNELYrjL47/4SVqxknhYcPY4tdOf2zvpVknguFH8UECwKc"
    "SL5eGDJR7WBTGQsBC1ZSUL6Othk6eLZClLFbH4G/lzWAU0ApmhsehcPO0+C5iBUmXhfDtBkMNVZc"
    "S74VbKXDBYqwQUaFXozYBBtOrgPmAz8tWFkCisri+LbQmWZ+d4fUPg5O+10EtgIZOBG3lgi33qDz"
    "0nm5Z9Kfk75pARgiEIMncw3oFpVEmLtmBX02W547nK2KMjOHiVrDHnrJLxrNYypx07csPOcznXP+"
    "UD4z86IafDBUBe2sYFrkN0U8mquJuhIv51nGF2yJvMP65PBz3Bwy6diIODdMNgfIxlwUptvg+YIE"
    "4FuUQLbxXaCeA3M5XAUpWBvtjQdahTBg+R16xyQfNsrcwOUGA5ApOIwHHSWJ6gpFUpXgydpEpoPA"
    "2Gvi8CcNnv2oI6Kb7HFa4BnaNc0LO2Wc6duXEXDoajFkPqIkpz2ItPx4TqrqjFFfxh67Ay6ipc8A"
    "7lRzAtXLPGTpdRPKtyMM3zoNkw4uAvcjhZImH5QtlaHBxyWhujgASfR0nDcvH1X5B8jVwBTc3MdP"
    "Xu5oAd7XqWisKoIP7otyp8u+xfclR2lFGgA9mEAm+yXxo3s8kslGzFXZjwMbqsjnzyb4K9YP+ze5"
    "wsw7ClgZFgSe13UJe6AaH+27BjVJy2mUVi++1YqEjaGHQbTqJRR5vWellZRh2DGZfCdjRPz2R+iV"
    "48X0FFMqO/mNWWNrg1v8zsAcFT3my+MZ3IBPGXBb9SbJ1G+BZyv0MTStGiz1emjEpxlwR0oap4xO"
    "Y+D7cNq6zhRtzG/ivS/5eF9CA5ZYZMFddT+WeyacUeizYsBBBPcqRKDUB1uiFfLPA9KzkDckhRVT"
    "vYeSNj9ICFNEO2J344fAU3hP8wSOv1LAwExpFJ+/H5RKbjNLluylQ3sRrlWLkIz/evHEuyO4xL4N"
    "g8/ksx7XVkPUuuWgfXGclLmYswpCu+g3C1ci/K2dLMoJhvNrjdmuZi1snLhF1o5EwpsFm9BV0wN8"
    "vW6CT0QE1X0Vgo00B3R4F+P90dfos00ErodOJ42uUXS5SjGOrltKM9U1cbLODJbP3YVRB95Qh74W"
    "2B9wAaZyHaHiVAg+iFKAMoERdBQeBc8lGWyi9yQJ3LUCvo09p/uamtBt00xul3YJri4Vxhh1TThQ"
    "fA4Wm+ZTmad9xDiqAOfnDLD0lxodm+EMGwObobBpE1z6FEn4BX6Z1kkJcxT2qlGv4/FA3S+zw/rG"
    "XI6zHqx88Ahn/TlCR860o8rUIvistw5ktg1gy6lk5Km5Cl81jODD2DYSZniPzPCawKOzksC7+S6V"
    "/XibnkudBYkLpeniBwpwjFrjixEz5viLeVj24i+demVGbbk5JM0aSWMxD3eaiRGp/eGAGu8fMg6v"
    "UuBs8Db4tdsLRQoUOBtj8uDiTA2M5vFGCZVB1IOH2K/TyTSwz8iBJAU0uzIHvdqn4Wq9ffglo5ra"
    "KfTgj9OXcdD4EmM+I5dcKJ5iBY9fAxF2L3K2BjPK51ncWDtI1ww+pBewCeo6ElA+yw7L535nYoMt"
    "ybq/JXjJVBgujSSB5NgmeuulH8CzdubQQCQrt84QpF4K4odTApzM5054eN5BtFn+g07kJ4B0UgO5"
    "gydwdtIGSImdw1kgtBzf8v1llz31hitJZXhovQ8RXzRI/oqmgOGDz+zSPdfhvZMYGl+fxuU9L0qG"
    "rIS4jUZ86HqhjER/KYEd3TmmUd4s8VT8SI5fqEa/u8KcLl0GQ7bH4yaHIijmY9l3p3VI2j1rwv32"
    "Gq5aymP3JR0837aK6Ck+owZ826BjchSUB1PA6Vk6LhlbCCf89oJc+mOyMC4eCgZr0XXgGyMluAye"
    "dTGQpyrPCb4rys34XUuOK5lAtok2FVmeBqLEF1FgI6xUFOKYeFVjakAo+NNgcFslRkaMPqHBymI8"
    "KnQdrjevA/GnIiSlopY58D0DbTYxYP1KGWKaTGGuTTQYlH6GkVUJoPEiDO3mp5DW9vv0/ZZrrNrZ"
    "Qrin+K/rp5+GhuYUcHPeB+F1s6inVR+JWk+4B/O2gthqJ+iSFMQFSRHw2SAS5q+2BZIfDrPUV5EU"
    "KQnO4xUpcGrTJfKtVJGZK6EOQqpXaejhQJh0mgVeN9fApr+aaNOxDXdePgwSc/zwbZMU/VNfQ8wW"
    "7UaYs53Sb6qQcSEGVB3y6X//GXJqF/DA81WVINPThIcra0DOyQfHFlDq3ezPODANaLi7jfWfKcGx"
    "7kNcfOoM/hH1xmULdGDlsotQLDRFxZtP4q8HYmgrG40i9qpEwzMZw5/tw4ctX4njHYCLT9MhTj8M"
    "11g8ojGVL/HqaQnkXzsDVR1H2HPiK/HF613QeiYfjbOukj32S2FUzAKeb73AtH1Lh9AXy9DobAvc"
    "HZABbXYWRyUtlk7sW2Fqfs6dnPkcAT2N56iElAmMfpOGT7Vf6wIjeWHfoUya8XkmKbCOgNH7ktCR"
    "q0nkV/KBpVRBvVuSFQ4K5dAtV5+RO1+BiMyZh/e/LEGvTxW4R3gBTnYbgG2FGlmWeB3Wb56OuxLn"
    "gkPqL5yyXIrfqgzhlfhvOiSlDb1zRLhRyVvI8pEWfGC3DEf3ZTEVa31g4d8qmjglAUd/62OTpDsm"
    "1QpzG1v+gJfhDhr64D9csuMYLZx1n8h2r4H+JQ/YPiN7VF29HbU8ZsNz7i+Qkx4gW/eZkEb325Qr"
    "uIIY7b5J5jshjuVIcvnyKR3NWQZBx6xhULwPxLMvY7GmCx5e/pqqjEbjK/tcNkqgFZqW3IUGncvs"
    "UVM/uPCMkuU1XJzdcQsP2x0Hy9zzqNHSAGabsrCgIh3df5fBAjJIU81nwpvCUHwj8IdU0xD68bMh"
    "9ZmwxrIj/mhwRxnH7v4lUas8Ub7AE5N2uLNt0rvg6A53PGC1ixg9MoD7l7xAXtmVMMXikDkiiCFp"
    "4qihro8b34dg0b/en/4wDITCXxD/cQkQKR6hWvEvyMFZG+Dz6EGQ4f2KKYeDYdGZByRERZ/unJZK"
    "V0U148A6XaAlcqQiQAA9xl/AtgJ/VttPg2x/nYMxpsrgY9RLzwfyYsMjMUxtzqQNbzPgRFIICHO5"
    "2LiZB+uUFuEP5np9ptQwDXyXinF831cE9Kbj6rB0sBPLJDdTb1HjaXbw0VkHop0mQHJdKozzzkCn"
    "H2m42TuYyap/AurWrmh8kTE9dDCZWtsOsM32rnDpxVbwTPhGAuwMSYeZJJRfvAKTudp4eelcTLB9"
    "QzX8IrBMpRVdj3xm5Lc7oo5HFJypzYLtqcfwweoYmlj2nQbLD9DAoCCYz5mOuemVYH4gAeb93AIh"
    "PsP0wR4d+HUSIPGTHyx+zgvqhuuQ8btEeG+8odmuznC8fD7X2EuRI3fHBM6lX4aem2rYzD8N9UPM"
    "8cesJ2SXymL6qtkSQ19k0cDFSXBf+DmsnNWOrlnSnC3vomlSnDtkQIKpkvEjmiPyglXSng/H80TJ"
    "65J3ptvE20kdLwdO2DeB2R4TKpl2C26Pv6KzJeS51r8/QqX7DSJcEQxbc+qw780j0nXxKKSeWcwd"
    "WbeBsy0mBYO278e7DsN09EIVbjQuwIcLhdHFMw42qJ0nigPnmCz2LBUS0UbxN154SK8bfy4TRN0w"
    "ISIzC/BPdigIbbgHS512E55PUUyc9yPI/JhE3D6JMxHeudTGWQJeyZvS0YnrJGNPJH1d2AnFTDs4"
    "Pt0Kb+pGcdxNiv6YMUKOt/Eytcbl5PZdN3SUKSR6YtuZW8GfaFlJLhH1FsB+uWz4/dWFlRijFIge"
    "6HifxohPkqS2JoO+e3ufNOoJ4rDWH+o5zxHVJBX/Zfs5TEwW5Oa//81I7ntFuPIrSZbNSaJlFIUx"
    "PmcgUNIH+u0lwaW0CkUvHoHViUl0q4srXLZIoSOuK/CR3yLsKT2L2w+vYX97StSf3lhAKn6YgG7g"
    "1/oDcnvw5wCLHim9MDaVDuPTU/Cq2nVoudbAbnGqpeDVRVoZhKWfplPlhs84eWqIShwVoksTN8DK"
    "dnXs940HL1uCV3ViKaurjQ62/EyV0j74pCIJwYLXyJqE+2xU+xDazzyJhu/00bYhHr3DX8OK7Pug"
    "F+9LRb8CHli0Bh6OAJR+2YAucafx0PTN5KNzKbTZxsGsjRmg6iLKKbnzL5M71CjrYILNPN7gfoQP"
    "NdtS6dylwhD4gRe65tlAjW81htu6k8nEKMIn5oGbs/VJueNa7HL8dxv1U6Ag8YIKPf1n63m+5GaR"
    "EtTb/GR9c7MY5ew4rHszF2OPxjGZYZ7odjgct6iyWGEWDMufPWMWB5XRpffb8fw9A9Kvkg5rrUZA"
    "XsAM1iR5kOnKF9ElfBfy5FwkHjJWkPkqhxz7c5Budf9V391ch3bxj0DH4QG2W9bR+0XytGH4Lub1"
    "pTJXt1rjrL+ziM8OeRwSvIIP11bXdy8YRKPCi+Br8RJ437WBa5UILvx7H2yd9rIFi/nxvvwF2Bwy"
    "m3tpwJUqm0VhcmYV/TuxkFqf24fX0tPhUqwxVeruhtInLF2xp5nSQ5TwNsnCTqEISLspwXVfUMAU"
    "Fuyiw8WPqdeYCmgoXWCt/EQ4SXuzQfpNFEqb/KFEbBl8/7ic7fnwDrWzTLHmYCQJsebjXgmVAjFJ"
    "PXROvgVcF6d6L+17QMuWwgv1HWT8nQG0/hDl2r7SxmvP3VDK6jOt6ghBKQcb6LfQg1VV8eC6OBYk"
    "TyxCFaOtcMAsA3elKULt1WjTE3F2+P2aHWyXmELeD2X0VEg5PeFTQ3nu6uLM3k5wiksGffNu2BER"
    "AV/kGOgrGqNdNrdQdegu/fLpLzvetRLFlT6wt4wQbun+QP1davBu3WXSJPidtEWdxbjYaPptUyRm"
    "lFuhyGw5GNSRgqtFJSglWADjnHE4rm+Of6WfwlpgaGR3JG672YHea+TJydorZHsCpS00CMSE5uGl"
    "D5ZkU7cdJruqwO+jWWjr9M20Bi3xedMH8kZ2HaRciQX304vRySsFB0ukOPsWldZfmzhBXR68Aovv"
    "F+j2Yl04sv4yeR+0HBrXe4Hqbnt06duDp6ovwUQvAZmK0vrdq5FEmAfhvckSWvLsE7kZMw3f3viL"
    "brOEUSEpFK7kNBNdbSmOAc8FOmeqGHKGljGHjS3gEB3G/hlnITAgBN5YJGL9MRWOjaQ1d35JOjwv"
    "uIeHrHYCKFyE0rgE4mbfjnszfOF2nKnp5DYgf4Uu07nh38jqzC487+9Hd65+yhbOnGBMzEKR92Qu"
    "dLpPh1aiDpPlddBhoMXNH7zCTsV+oZ2jnvTiTy4Kbwml3+fVYsCYCvbOXc+W7HjHdMqZ4bogLUyd"
    "e51IbpsNGXsEMdMtF9QvlVEpq8oV/QvnsuVvLzInxNRgpfNetl+gGBUt35BfYvU4YyQTfU9cotaC"
    "JdRYqxqz4p+C1vO/uPVFKcovG0dB9ixG50VQ/oFdODiUhBFfzmPwx0hY05IEQsFmUO9dg+0qt2Hs"
    "0hb6+9ha2nsqHPaXh+IGkQl8a5UFaonD9RGd5eSuRTgcd4mHV8Uy9FTHcmKb6ojr3l7En69XE8HN"
    "tszn8jx0aB0lKV4OZHpzLDiHn8U1TnYwc7UgE3qiGJuNz6KnVSK5+TuL+Pu2QOyWLCyM8SSfohWQ"
    "x2kBGLwIA40lPPBbIgmvvXQCe4MFEMDzH8v90QLpPDyM/IlUHDrkBz+vdpFdsw7gXg9/MlIRSETH"
    "h4njfCUsihSGl3a9EL5mLir+ykBzuWfgeCDFdEHT5vqbhuvxhHM+O2S/CsZKik1vJfykXQiQ/Tgd"
    "lTv5sfulAmqd/ck68rdBSNR5LAgRBRUpU5SdqQyx0ZZw/VUYeZYTCrq+XbBGOBBPJwTgwUYjUnzb"
    "BCbP9ZDBTXYkTciUeu4NBDcwI0XTo6EhfRp3u/K/7tGWBqljH+nz6y1oF2DMQOQMjPq1HMdTwtla"
    "LSHsaU8gsSot5M3K97BEOgMUonRBus8e2iYUYPJzIgjlBeNvp2Vk1m5xboG6Db7SnsENO38C8h+1"
    "kI0vM/Djw3Bw8eGF+c9/0gqayaiZnGfHOhLgv9Vv6MtVh+gpByXw1uXD3S7CYLDpBu0/f4HsKT2H"
    "J+4rQO1ED51IcYfWxuOUGNWAztpsMvn8EAZZbgCpDcvpq44hVjUrBsxrJNBrtRS4/kkAgRWrSZrt"
    "LI6xsjbhNZLkHGzIZXO+J+Cn0t+0Z8dB2N0TQ86lVpOakV/E43AIvei7DD4ejqSrxguRN8kG6+cR"
    "KCzMxYPoiY4LO3B6lAe0WA3TMBk3WGKhTeVu8dNjH73gZsEMEONZQwSCz5IzTuKY68zHuviuR3Vx"
    "GXx5KxFch1Oh7G4/XftwGxo/04KxhH/Zu3oRx76sBDuVb+FB/YOwV/c+9QouRLsls7n7m52g4dQD"
    "kAuKpc9t0umOXy3I3xaNPK7faH1jHGEMNOjZ4Bs4QyqcOXRfgUTF1MJ6nhpoiLmKzceNSJj1cnCs"
    "vo6Oa4fJsmP64Jb8isxpJXR9TTxONd0Dq8UOMBQgCFccH4NstgLojm3H4y2JaNI+BFb0ArHkf0IU"
    "+SzZUa35+MkoHDcmBdD2OUMkuscVwwVi2JEtK5Go8nAVRQuYo9U8HF2TSGr5KxvKtTLJGrkQGmaV"
    "Z+oQEozWjeUw/8IVfOIVgV9za2iUag9d1VkFYp+twMBvLpXZv5EY54+SRwZ+4DjfHouHh8DnnQAk"
    "ZWSRaJ44OJqmimZmZSRFwwflT86AuznmcCBKB66WZ5L1jm40bE8NZXkLsWyYH3d161JxWYoyvnPg"
    "xpESNLYppjOzj9Ge+T1s5MlT4C51jJhc1MA/KZbk67endD93HlflQSHrb3UWcgt6QUb/GrIRjzHv"
    "mQ3+fN9GFMzi633uXYSNk4PMYv2jmPzjF70zfycxP+sC6Q/dYM2tYpwbmEhPzp8Nv4LnwoBRCVVq"
    "fYgNLldAp7kXTyxRIMnKCnBodCGMRzXAwsFEqqnK1pnp3EeNiFfoOvslrigKZbkrJMnSRbfwr/Fc"
    "qErYD+JpkiTtkge5yZ9PL3VMo1NzF6K9+zyu5FYx8tji36bxEOAMvtlPa/fJkrjjDjQ6RQh9xu6R"
    "LJ9ejDhUV+8wRx8ODIYC/OeOH7X84c6T86BkeBeE3+4D4QPp5HDgbNZmXQ7ZW3iW8N3oxiNZksCj"
    "WUpllnSQ7bsq4PbWDtjSv4+mhUej+bNnWPn3I/O1awDtEuXh2mxVPP/BnmkRnKQWfcUkTV4b5Nao"
    "kxO74+vl2+Ux2fufcYZ/kh6ff/57roap6aeh9LwsZ2rxBK2dlYlvu/pgVs5RmvmZ4tDf02T/B1uT"
    "gzGOUNE5XN853kLDz60B2zMUsq15qPnUI7y7XwkbwiPwQJQCXU3r6FfrTGx1fIpuzxbSbdOLMfuf"
    "h/qUt2Oojj7unuUI3yo7cd7dKCgfd8DLf5PR5Bi/afBJCe547yWQux+MT0znQPFVIDM0X7EvKudC"
    "97gtDit+hsDRPDq5NA1lpbW4a7UWc1q9+Zk9OTZ0kfA9qv6glK3q8cEFOndgzyEJXOhSCuVzqsh4"
    "ZzMb76FM6aWQOrb2D3J0gzDC9z7Gc88R/pIKzNf0BOMieU7O70am0G4R8MB/aDHwHaepn0R+TRkc"
    "ljHABTuFMLJdHOom0+icg1b4VTYJunwqSITuKHGU2gb1EpF02Q0zaLWvgFC3fvw97yz0HR3EZfJN"
    "NG2VLvv890+ceO1GJwpug6LWcbhWJcJ5dOQHBkkn4+Z+Pu78jY+JteVFNiBuNzZ+qCFht4ow9Ls4"
    "ufy1kw19MkXiZYsZk0ph6n1kBte3ohinonKh1fQpe+XMLdQrDSKY30TLC26TEBFpyHocSYK+UiL9"
    "vhxeLdlkerWfBYP54uCV+JbJ9lbH4YbLdNWnSXrZrI2cswyhuRu0YUf8foxethsSjgQwO0rzKW+v"
    "FXut+TieUr6CNw5UA19xMHFg+HDOkr2QMQmooBgPC57aYciDQPprtxi1kObnuHxOZZ4YmeJ7Hh+Y"
    "5XSUOrI9rHMVUKWzubC1aTv7+OxDkNsvw+U9oE2eGuohz3xKdf3dmDRTO3hjnAQSa6Vgb8hNXHVq"
    "HLPYXOxw/ksNNRhaviMCI7ptYU8MAVHDXnj7YbKu2k0HCyPDwLn1P/KfmjJovVIBp2JnfL7jEDUR"
    "lEC1T324eosd2/t7L2w2WwYZEr1UBBYzA+ap/346F5V6bmCjTQyIS+nhqeYv9PvbT4yKwyY46bUY"
    "s4UaYTDsOFl+pZSMZphi9MhsrPSMoZZ9ipy7o/MxOCCSKrVV0F4rNSxV6AD7lDI8oS4Ce3OSQXT6"
    "KuJkwI+xq19Dk/U/V806DXFDD+ksTWvIFawkVfz3aFynAaRGZ7IySYNwqi4HhJKWgvXkDVj3p5is"
    "bdgKcsklOCbej4POqrDsykOS6eJB+p944OzOeZgcUkauVaxicy3CYP1UMZXP8QbN3DmcyYmLlDsu"
    "SjVKw0HzYRl2bH39z1B1kL43FP+K3CNhpzIwfbwa90Ynw8k1Xeye7Waw20ETGOHH1UtNN5AmlSd1"
    "S3g6kOfSftD9pA9ZI6fJxBdlGFkmSFo2VaDk6Y+sZAAX/D1L6DDPDO7SyGLCWqhR5tNjRsr3JbV5"
    "fgnOVVfiqiz3+reDZyHEYxI3rzIixrPfsdtHSiHRbCuF8kI86zmKqVW5bL3QcajeuQs2zJtmajHz"
    "I5zjUQTnytu06oQNnn+mSX2fzoSYx6uppZIwRmgPmwpfkOQsWHGXeGo9wcbbD03ljhxiKlxNcd7T"
    "6ZxiVSc4sU6C1P/2o95fc/DtvXOopFsB8V/0OQ/nqnHzxMrANvMSEclaCoH5BM+4tMJ20TBUS9nE"
    "Tmy+zaz3/0V/Z5rA2uve+E2xgzj0srS5IIMoyxaTftktsJb3Ktjb+oCvkyaUPLyAHht+0Hx3Fbiq"
    "EAohr+Pxxu8YeLd3OafD1x/25Z1mXLUT6JTVbPhSMg/m9+kTvXt8ZHKzKh43MEOpIFfo+mYCK623"
    "YHzqFrZQpJbUeejCr6F0+n7aaUxRWIXxWhFUiycA9TZZMrXX1WCONC/H+3gf+fvWGMrs8ilf9G1c"
    "sFaUO/hYjH3cZIsPj/eYnhZJwoION7DQjAK/vr/ELY2Sgehy6mpozurF8HE7AtvrrbZQUrJYBI4M"
    "VuJOrSrSmvMBV/16iuc0eKFePoD4yFuTzY3FpFVyqv7DCS8o7ZHCXsplBe+fxnkv5GiC2HZSXHmH"
    "amfpQv3FcAjZKEc8+E7CIos5tDmvGvQzjtCe7nqGfdZMv39PQp1kHnzfhMy/qCZx26ZzBXm0mC0m"
    "52FFXByZezoM3ypfo5ebS2DrnrUkXOkoLR++RrM4hYyAXwieFNde8fDWSrx/5Aw0SZWC/uFwxuVn"
    "NJ38mQh5uTowp9cfAsoEOV3sGXokgUsauqdImUU3rc44+W8TxNDezwQOCSQwyX+NcG9lM06syKe5"
    "Pv0Yo1OGV6u08YKdEVruPkRuHo9F2SPlOPTSAIq0KR5faERKn8qRqFsc0H4yGyUctoPj+HIyHNvC"
    "8Ad9wcM5dSTXtQV2pKZDbPEEbQp5iCM5/Sznhi63aGkdWGkuwGyBIABZRTi+QQJKBNzw0/s+kn1S"
    "hSy+oowvrCjk/A0nN7IM6P7qA/R5QRMk1mRCkuge+sHaHnZvKIJHT1eS3KiN9MKqfUy6mS9dLJVC"
    "jY8tAaUj+4nzDcSguebk+vNMSi5ow4sN8+DZRSfoEavFoE2yHGcdS/Ap4uesCNhG0qSXwadEH5S1"
    "B2i92sZc+SqNc713ofMrPdDgq0PIT8O8dmcSO/sMvDuoxx73EMcrzL/9KDaTI9B4D1r/cFi5thkQ"
    "/7IWndRncOym6smmMC5KZ+uZWjDe8HbNVrJhG8WjB2eg8vZlsFJOB7fZd0FbTpfp0fLrdFaVCmmZ"
    "85iWafhhVLE/ifUX5B4NVqZrT6Yx1rbTic9oIXwfamfEhiZw/N1XTPvkDeU+0zi5Lca4WL2Lrh4R"
    "hm8Lk9HhihM6JPwEH8831DaxBu8uDYfWc9k4tiMUz5wWAu0oF7D3PkK61BPQLfoB+3DqMT2Zr45V"
    "CWNsRW46DHEuoQVPPcX/imjakCb3lNwjckufASeFEObYgzOgx9tF7Sd76PGGuXBTvqj2Ty9D5abs"
    "yJRQJKq3W1P2dA9xqgtBjfn1pKTtLByJmKjrEgoH//BkahT4HLUhBsXPlrJeH0LrozrGQPe/aDzw"
    "5QzezQxiQ/YHkKrzBeSeBcsMfFvAfa/ZSfQFrNnVUy74J66HXLF2wr81smBQ3Ezsjxrhq98i3PLe"
    "MzjTrgJi3hwFLZ/9YP9pAei79ZJ5U6chV20zuO3ug1+Ga6DtVSZ9yqpggpoKqHt9xVuFopi8uRZ0"
    "mvKRt+8zTR+3MDVR78Nbdneg4cBieMznD35CZST9qAWdX3cOiEsapi2SoI/4IplT5dY0yt2M7DOy"
    "Q9/gUjK08x0x6/HDcx52+CPwJXNI/p91Ao7Xy5pl4c8TTqTcS5TR+BQOc5Zb4+9/fn0TWkCdFh9i"
    "388/RP4H7LyavQ==")
_MU0 = np.frombuffer(
    _zlib.decompress(_b64.b64decode(_MU0_B64)), dtype=np.float32
).reshape(_M + 1, _P).copy()


def _dg(row, mat, precision=_HI):
    return jax.lax.dot_general(
        row, mat, (((1,), (1,)), ((), ())),
        precision=precision, preferred_element_type=jnp.float32)


def _s2v_kernel(z_ref, a_ref, b_ref, c_ref, th1_ref, th2_ref, th3_ref,
                th2c_ref, th3c_ref, th4a_ref, th4b_ref, mu0_ref, out_ref):
    z = z_ref[0, 0]
    b1 = b_ref[1, 0]
    b2 = b1 * b1

    xp = jnp.concatenate(
        [a_ref[...], c_ref[...], jnp.zeros((5, _NF), jnp.float32)], axis=0)
    gram = _dg(xp, xp)

    rid = jax.lax.broadcasted_iota(jnp.int32, (16, 16), 0)
    cid = jax.lax.broadcasted_iota(jnp.int32, (16, 16), 1)
    nb_mask = (cid < _M) & (cid != rid)
    w = gram + b2
    pos = jnp.sum(jnp.where(nb_mask, jnp.maximum(w, 0.0), 0.0), axis=1,
                  keepdims=True)
    neg = jnp.sum(jnp.where(nb_mask, jnp.maximum(-w, 0.0), 0.0), axis=1,
                  keepdims=True)

    g9 = jnp.sum(jnp.where((rid == _M) & (cid == 9), gram, 0.0))
    g8 = jnp.sum(jnp.where((rid == _M) & (cid == 8), gram, 0.0))
    vcol = jax.lax.broadcasted_iota(jnp.int32, (16, 1), 0)
    wlast = jnp.where(vcol == 9, g8, g9)

    arows = jax.lax.broadcasted_iota(jnp.int32, (16, 1), 0)
    a_z = jnp.sum(jnp.where(arows == z, xp, 0.0), axis=0, keepdims=True)
    t1 = _K * _dg(a_z, th1_ref[...])

    th3 = th3_ref[...]
    th3c = th3c_ref[...]
    base = (t1
            + _K * (pos * jnp.maximum(th3, 0.0)
                    + neg * jnp.maximum(-th3, 0.0))
            + _K * (jnp.maximum(wlast, 0.0) * jnp.maximum(th3c, 0.0)
                    + jnp.maximum(-wlast, 0.0) * jnp.maximum(-th3c, 0.0)))

    th2 = _K * th2_ref[...]
    th2t_lo = jnp.transpose(th2[:, :128])
    th2_hi = th2[:, 128:]
    th2c = th2c_ref[...]

    def t2mv(u):
        t2m = _dg(u[:, 128:], th2_hi, _LO)
        u_col = jnp.reshape(u[:, :128], (128, 1))
        t2v = jnp.sum(th2t_lo * u_col, axis=0, keepdims=True)
        return t2m + t2v
    cols = [mu0_ref[v:v + 1, :] for v in range(_M + 1)]
    s_all = functools.reduce(jnp.add, cols[:_M])
    for _ in range(_T):
        bc = base + _K * _dg(cols[_M], th2c)
        sv = s_all - cols[0]
        for v in range(_M):
            x = jnp.maximum(bc[v:v + 1, :] + t2mv(sv), 0.0)
            s_all = sv + x
            cols[v] = x
            if v + 1 < _M:
                sv = s_all - cols[v + 1]
        cols[_M] = jnp.maximum(bc[_M:_M + 1, :] + t2mv(s_all), 0.0)

    r4a = _dg(s_all, th4a_ref[...])
    mu_z = functools.reduce(
        jnp.add,
        [jnp.where(z == v, 1.0, 0.0) * cols[v] for v in range(_M)])
    r4b = _dg(mu_z, th4b_ref[...])
    out_ref[...] = jnp.concatenate([r4a, r4b], axis=1)


@jax.jit
def kernel(A, b, c, z, theta1, theta2, theta3, theta2c, theta3c, theta4a,
           theta4b):
    out = pl.pallas_call(
        _s2v_kernel,
        out_shape=jax.ShapeDtypeStruct((1, 2 * _P), jnp.float32),
    )(z.reshape(1, 1), A.reshape(_M, _NF), b.reshape(2, 1), c,
      theta1, theta2, theta3.reshape(1, _P),
      theta2c, theta3c.reshape(1, _P), theta4a, theta4b, jnp.asarray(_MU0))
    return out
